# baseline (speedup 1.0000x reference)
import functools
import jax
import jax.numpy as jnp
from jax import lax
from jax.experimental import pallas as pl
from jax.experimental.pallas import tpu as pltpu
from jax.experimental.pallas import tpu_sc as plsc

B, C, D = 128, 8192, 64
NW = 32
B_SC = 64
B_TC = B - B_SC
BPW = B_SC // NW
NCB = C // 128
CBC = 4
NCHUNK = NCB // CBC

_mesh = plsc.VectorSubcoreMesh(core_axis_name="c", subcore_axis_name="s")


@functools.partial(
    pl.kernel,
    out_type=[
        jax.ShapeDtypeStruct((B_SC, D), jnp.float32),
        jax.ShapeDtypeStruct((NW, 16), jnp.int32),
    ],
    mesh=_mesh,
    scratch_types=[
        pltpu.VMEM((8, CBC, 1024), jnp.float32),
        pltpu.VMEM((8, CBC, 1024), jnp.float32),
        pltpu.VMEM((8, 1, 1024), jnp.float32),
        pltpu.VMEM((D,), jnp.float32),
        pltpu.VMEM((16,), jnp.int32),
        pltpu.SemaphoreType.DMA,
        pltpu.SemaphoreType.DMA,
    ],
    compiler_params=pltpu.CompilerParams(needs_layout_passes=False,
                                         use_tc_tiling_on_sc=False),
)
def _sc_run(xp_hbm, rows_out, idx_out, chunk_a, chunk_b, rowbuf_v,
            stage_v, win_v, sem_a, sem_b):
    cid = lax.axis_index("c")
    sid = lax.axis_index("s")
    wid = sid * 2 + cid
    lane = lax.iota(jnp.int32, 16)
    winvec = jnp.zeros((16,), jnp.int32)

    def process(chunk, cbase, mv, mi):
        @plsc.parallel_loop(0, CBC, unroll=2, carry=(mv, mi))
        def cb_body(cb, carry):
            mv, mi = carry
            for g8 in range(8):
                cl0 = g8 * 16
                a0 = jnp.zeros((16,), jnp.float32)
                a1 = jnp.zeros((16,), jnp.float32)
                a2 = jnp.zeros((16,), jnp.float32)
                a3 = jnp.zeros((16,), jnp.float32)
                for dt in range(8):
                    for di in range(0, 8, 4):
                        v0 = chunk[dt, cb, pl.ds(di * 128 + cl0, 16)]
                        v1 = chunk[dt, cb, pl.ds((di + 1) * 128 + cl0, 16)]
                        v2 = chunk[dt, cb, pl.ds((di + 2) * 128 + cl0, 16)]
                        v3 = chunk[dt, cb, pl.ds((di + 3) * 128 + cl0, 16)]
                        a0 = a0 + v0 * v0
                        a1 = a1 + v1 * v1
                        a2 = a2 + v2 * v2
                        a3 = a3 + v3 * v3
                acc = (a0 + a1) + (a2 + a3)
                cls = (cbase + cb * 128 + cl0) + lane
                upd = acc > mv
                mv = jnp.where(upd, acc, mv)
                mi = jnp.where(upd, cls, mi)
            return mv, mi

        return cb_body

    for bi in range(BPW):
        b = wid * BPW + bi

        pltpu.async_copy(xp_hbm.at[pl.ds(b * 8, 8), pl.ds(0, CBC), :],
                         chunk_a, sem_a)

        def pair_body(j, carry):
            mv, mi = carry
            c0 = 2 * j
            pltpu.async_copy(
                xp_hbm.at[pl.ds(b * 8, 8), pl.ds((c0 + 1) * CBC, CBC), :],
                chunk_b, sem_b)
            pltpu.make_async_copy(
                xp_hbm.at[pl.ds(b * 8, 8), pl.ds(c0 * CBC, CBC), :],
                chunk_a, sem_a).wait()
            mv, mi = process(chunk_a, c0 * CBC * 128, mv, mi)

            @pl.when(j < NCHUNK // 2 - 1)
            def _():
                pltpu.async_copy(
                    xp_hbm.at[pl.ds(b * 8, 8), pl.ds((c0 + 2) * CBC, CBC), :],
                    chunk_a, sem_a)

            pltpu.make_async_copy(
                xp_hbm.at[pl.ds(b * 8, 8), pl.ds((c0 + 1) * CBC, CBC), :],
                chunk_b, sem_b).wait()
            mv, mi = process(chunk_b, (c0 + 1) * CBC * 128, mv, mi)
            return mv, mi

        maxv, maxi = lax.fori_loop(
            0, NCHUNK // 2, pair_body,
            (jnp.full((16,), -1.0, jnp.float32), jnp.zeros((16,), jnp.int32)))

        gmax = jnp.max(maxv)
        winner = jnp.min(jnp.where(maxv == gmax, maxi, C))
        winvec = jnp.where(lane == bi, winner, winvec)

        cbw = winner >> 7
        clw = winner & 127
        pltpu.sync_copy(xp_hbm.at[pl.ds(b * 8, 8), pl.ds(cbw, 1), :],
                        rowbuf_v)
        zero16 = jnp.zeros((16,), jnp.int32)
        for s in range(4):
            d = lane + s * 16
            dtv = d >> 3
            wv = (d & 7) * 128 + clw
            vs = plsc.load_gather(rowbuf_v, [dtv, zero16, wv])
            stage_v[pl.ds(s * 16, 16)] = vs
        pltpu.sync_copy(stage_v, rows_out.at[b])

    win_v[...] = winvec
    pltpu.sync_copy(win_v, idx_out.at[wid])


def _tc_body(xt_ref, masked_ref, idx_ref):
    xv = xt_ref[0]
    ss2 = jnp.sum(xv * xv, axis=0, keepdims=True)
    maxv = jnp.max(ss2)
    iot = jax.lax.broadcasted_iota(jnp.int32, (1, C), 1)
    idx_s = jnp.min(jnp.where(ss2 >= maxv, iot, C))
    mask = (iot == idx_s).astype(jnp.float32)
    row = jnp.sum(xv * mask, axis=1, keepdims=True)
    masked_ref[0] = jnp.broadcast_to(row, (D, 128))
    idx_ref[0] = jnp.full((8, 128), idx_s, jnp.int32)


def _tc_run(xt_tail):
    return pl.pallas_call(
        _tc_body,
        grid=(B_TC,),
        in_specs=[pl.BlockSpec((1, D, C), lambda i: (i + B_SC, 0, 0))],
        out_specs=[
            pl.BlockSpec((1, D, 128), lambda i: (i, 0, 0)),
            pl.BlockSpec((1, 8, 128), lambda i: (i, 0, 0)),
        ],
        out_shape=[
            jax.ShapeDtypeStruct((B_TC, D, 128), jnp.float32),
            jax.ShapeDtypeStruct((B_TC, 8, 128), jnp.int32),
        ],
    )(xt_tail)


def kernel(x):
    xp = (x.reshape(B, NCB, 128, 8, 8)
          .transpose(0, 3, 1, 4, 2)
          .reshape(B * 8, NCB, 1024))
    xt = x.transpose(0, 2, 1)

    rows_sc, idx16 = _sc_run(xp)
    masked_tc, idxb_tc = _tc_run(xt)

    idx_sc = idx16[:, :BPW].reshape(B_SC)
    idx_tc = idxb_tc[:, 0, 0]
    masked = jnp.concatenate(
        [rows_sc, masked_tc[:, :, 0]], axis=0).reshape(B, 1, D)
    idx = jnp.concatenate([idx_sc, idx_tc], axis=0)
    return (masked, idx, idx)


# speedup vs baseline: 2.2880x; 2.2880x over previous
import functools
import jax
import jax.numpy as jnp
from jax import lax
from jax.experimental import pallas as pl
from jax.experimental.pallas import tpu as pltpu
from jax.experimental.pallas import tpu_sc as plsc

B, C, D = 128, 8192, 64
NW = 32
B_SC = 64
B_TC = B - B_SC
BPW = B_SC // NW
NCB = C // 128
CBC = 4
NCHUNK = NCB // CBC

_mesh = plsc.VectorSubcoreMesh(core_axis_name="c", subcore_axis_name="s")


@functools.partial(
    pl.kernel,
    out_type=[
        jax.ShapeDtypeStruct((B_SC, D), jnp.float32),
        jax.ShapeDtypeStruct((NW, 16), jnp.int32),
    ],
    mesh=_mesh,
    scratch_types=[
        pltpu.VMEM((8, CBC, 1024), jnp.float32),
        pltpu.VMEM((8, CBC, 1024), jnp.float32),
        pltpu.VMEM((8, 1, 1024), jnp.float32),
        pltpu.VMEM((D,), jnp.float32),
        pltpu.VMEM((16,), jnp.int32),
        pltpu.SemaphoreType.DMA,
        pltpu.SemaphoreType.DMA,
    ],
    compiler_params=pltpu.CompilerParams(needs_layout_passes=False,
                                         use_tc_tiling_on_sc=False),
)
def _sc_run(xp_hbm, rows_out, idx_out, chunk_a, chunk_b, rowbuf_v,
            stage_v, win_v, sem_a, sem_b):
    cid = lax.axis_index("c")
    sid = lax.axis_index("s")
    wid = sid * 2 + cid
    lane = lax.iota(jnp.int32, 16)
    winvec = jnp.zeros((16,), jnp.int32)

    def process(chunk, cbase, mv, mi):
        @plsc.parallel_loop(0, CBC, carry=(mv, mi))
        def cb_body(cb, carry):
            mv, mi = carry
            for g8 in range(8):
                cl0 = g8 * 16
                a0 = jnp.zeros((16,), jnp.float32)
                a1 = jnp.zeros((16,), jnp.float32)
                a2 = jnp.zeros((16,), jnp.float32)
                a3 = jnp.zeros((16,), jnp.float32)
                for dt in range(8):
                    for di in range(0, 8, 4):
                        v0 = chunk[dt, cb, pl.ds(di * 128 + cl0, 16)]
                        v1 = chunk[dt, cb, pl.ds((di + 1) * 128 + cl0, 16)]
                        v2 = chunk[dt, cb, pl.ds((di + 2) * 128 + cl0, 16)]
                        v3 = chunk[dt, cb, pl.ds((di + 3) * 128 + cl0, 16)]
                        a0 = a0 + v0 * v0
                        a1 = a1 + v1 * v1
                        a2 = a2 + v2 * v2
                        a3 = a3 + v3 * v3
                acc = (a0 + a1) + (a2 + a3)
                cls = (cbase + cb * 128 + cl0) + lane
                upd = acc > mv
                mv = jnp.where(upd, acc, mv)
                mi = jnp.where(upd, cls, mi)
            return mv, mi

        return cb_body

    for bi in range(BPW):
        b = wid * BPW + bi

        pltpu.async_copy(xp_hbm.at[pl.ds(b * 8, 8), pl.ds(0, CBC), :],
                         chunk_a, sem_a)

        def pair_body(j, carry):
            mv, mi = carry
            c0 = 2 * j
            pltpu.async_copy(
                xp_hbm.at[pl.ds(b * 8, 8), pl.ds((c0 + 1) * CBC, CBC), :],
                chunk_b, sem_b)
            pltpu.make_async_copy(
                xp_hbm.at[pl.ds(b * 8, 8), pl.ds(c0 * CBC, CBC), :],
                chunk_a, sem_a).wait()
            mv, mi = process(chunk_a, c0 * CBC * 128, mv, mi)

            @pl.when(j < NCHUNK // 2 - 1)
            def _():
                pltpu.async_copy(
                    xp_hbm.at[pl.ds(b * 8, 8), pl.ds((c0 + 2) * CBC, CBC), :],
                    chunk_a, sem_a)

            pltpu.make_async_copy(
                xp_hbm.at[pl.ds(b * 8, 8), pl.ds((c0 + 1) * CBC, CBC), :],
                chunk_b, sem_b).wait()
            mv, mi = process(chunk_b, (c0 + 1) * CBC * 128, mv, mi)
            return mv, mi

        maxv, maxi = lax.fori_loop(
            0, NCHUNK // 2, pair_body,
            (jnp.full((16,), -1.0, jnp.float32), jnp.zeros((16,), jnp.int32)))

        gmax = jnp.max(maxv)
        winner = jnp.min(jnp.where(maxv == gmax, maxi, C))
        winvec = jnp.where(lane == bi, winner, winvec)

        cbw = winner >> 7
        clw = winner & 127
        pltpu.sync_copy(xp_hbm.at[pl.ds(b * 8, 8), pl.ds(cbw, 1), :],
                        rowbuf_v)
        zero16 = jnp.zeros((16,), jnp.int32)
        for s in range(4):
            d = lane + s * 16
            dtv = d >> 3
            wv = (d & 7) * 128 + clw
            vs = plsc.load_gather(rowbuf_v, [dtv, zero16, wv])
            stage_v[pl.ds(s * 16, 16)] = vs
        pltpu.sync_copy(stage_v, rows_out.at[b])

    win_v[...] = winvec
    pltpu.sync_copy(win_v, idx_out.at[wid])


def _tc_body(xt_ref, masked_ref, idx_ref):
    xv = xt_ref[0]
    ss2 = jnp.sum(xv * xv, axis=0, keepdims=True)
    maxv = jnp.max(ss2)
    iot = jax.lax.broadcasted_iota(jnp.int32, (1, C), 1)
    idx_s = jnp.min(jnp.where(ss2 >= maxv, iot, C))
    mask = (iot == idx_s).astype(jnp.float32)
    row = jnp.sum(xv * mask, axis=1, keepdims=True)
    masked_ref[0] = jnp.broadcast_to(row, (D, 128))
    idx_ref[0] = jnp.full((8, 128), idx_s, jnp.int32)


def _tc_run(xt_tail):
    return pl.pallas_call(
        _tc_body,
        grid=(B_TC,),
        in_specs=[pl.BlockSpec((1, D, C), lambda i: (i + B_SC, 0, 0))],
        out_specs=[
            pl.BlockSpec((1, D, 128), lambda i: (i, 0, 0)),
            pl.BlockSpec((1, 8, 128), lambda i: (i, 0, 0)),
        ],
        out_shape=[
            jax.ShapeDtypeStruct((B_TC, D, 128), jnp.float32),
            jax.ShapeDtypeStruct((B_TC, 8, 128), jnp.int32),
        ],
    )(xt_tail)


def kernel(x):
    xp = (x.reshape(B, NCB, 128, 8, 8)
          .transpose(0, 3, 1, 4, 2)
          .reshape(B * 8, NCB, 1024))
    xt = x.transpose(0, 2, 1)

    rows_sc, idx16 = _sc_run(xp)
    masked_tc, idxb_tc = _tc_run(xt)

    idx_sc = idx16[:, :BPW].reshape(B_SC)
    idx_tc = idxb_tc[:, 0, 0]
    masked = jnp.concatenate(
        [rows_sc, masked_tc[:, :, 0]], axis=0).reshape(B, 1, D)
    idx = jnp.concatenate([idx_sc, idx_tc], axis=0)
    return (masked, idx, idx)
